# Initial kernel scaffold; baseline (speedup 1.0000x reference)
import jax
import jax.numpy as jnp
from jax.experimental import pallas as pl


def kernel(x, edge_index, edge_weight, W1, W2, Wp, bp, perm1, perm2):
    raise NotImplementedError("write your pallas kernel here")


# speedup vs baseline: 16.3233x; 16.3233x over previous
import dataclasses

import jax
import jax.numpy as jnp
from jax import lax
from jax.experimental import pallas as pl
from jax.experimental.pallas import tpu as pltpu
from jax.experimental.pallas import tpu_sc as plsc

N = 10000
D = 128
E = 320000
C = 128
NT = 16
NCHUNK = 160
EPAD = NT * NCHUNK * C
WCH = EPAD // (32 * C)
SB = 624
SREM = N - SB * NT

_mesh = plsc.VectorSubcoreMesh(core_axis_name="c", subcore_axis_name="s")
_f32 = jnp.float32
_i32 = jnp.int32

_sc_params = pltpu.CompilerParams()
if "needs_layout_passes" in pltpu.CompilerParams.__dataclass_fields__:
    _sc_params = dataclasses.replace(_sc_params, needs_layout_passes=False)


def _deg_body(dstd, ewd, zeros1, degp, dstb, ewb, degb, acc, sem):
    c = lax.axis_index("c")
    s = lax.axis_index("s")
    w = s * 2 + c
    pltpu.sync_copy(dstd.at[w], dstb)
    pltpu.sync_copy(ewd.at[w], ewb)

    @pl.when(s == 0)
    def _():
        pltpu.sync_copy(zeros1, degb)
        pltpu.sync_copy(degb, acc)

    plsc.subcore_barrier()

    @pl.loop(0, WCH)
    def _(j):
        pltpu.async_copy(ewb.at[j], acc.at[dstb.at[j]], sem, add=True)

    @pl.loop(0, WCH)
    def _(j):
        pltpu.make_async_copy(ewb.at[0], acc.at[dstb.at[0]], sem).wait()

    plsc.subcore_barrier()

    @pl.when(s == 0)
    def _():
        pltpu.sync_copy(acc, degb)
        pltpu.sync_copy(degb, degp.at[pl.ds(c * N, N)])


_deg_call = pl.kernel(
    _deg_body,
    out_type=jax.ShapeDtypeStruct((2 * N,), _f32),
    mesh=_mesh,
    scratch_types=[
        pltpu.VMEM((WCH, C), _i32),
        pltpu.VMEM((WCH, C), _f32),
        pltpu.VMEM((N,), _f32),
        pltpu.VMEM_SHARED((N,), _f32),
        pltpu.SemaphoreType.DMA,
    ],
)


def _mm_body(x_ref, w_ref, h_ref):
    h_ref[...] = jnp.dot(x_ref[...], w_ref[0], preferred_element_type=_f32)


_MMB = 1000


def _mm_call(x, ws):
    return pl.pallas_call(
        _mm_body,
        grid=(2, N // _MMB),
        in_specs=[
            pl.BlockSpec((_MMB, D), lambda q, i: (i, 0)),
            pl.BlockSpec((1, D, D), lambda q, i: (q, 0, 0)),
        ],
        out_specs=pl.BlockSpec((_MMB, D), lambda q, i: (q * (N // _MMB) + i, 0)),
        out_shape=jax.ShapeDtypeStruct((2 * N, D), _f32),
    )(x, ws)


def _dinv_body(deg_ref, dinv_ref):
    dsum = deg_ref[0, :] + deg_ref[1, :]
    dinv_ref[...] = lax.rsqrt(jnp.clip(dsum, 1e-6, None)).reshape(1, -1)


def _dinv_call(deg2):
    return pl.pallas_call(
        _dinv_body,
        grid=(1,),
        in_specs=[pl.BlockSpec((2, N), lambda i: (0, 0))],
        out_specs=pl.BlockSpec((1, N), lambda i: (0, 0)),
        out_shape=jax.ShapeDtypeStruct((1, N), _f32),
    )(deg2)


def _norm_body(srcd, dstd, ewd, dinv, pstack, normd, pidxd,
               srcb, dstb, ewb, p1o, p2o, dinvb, p1b, p2b):
    c = lax.axis_index("c")
    s = lax.axis_index("s")
    w = s * 2 + c
    pltpu.sync_copy(srcd.at[w], srcb)
    pltpu.sync_copy(dstd.at[w], dstb)
    pltpu.sync_copy(ewd.at[w], ewb)
    pltpu.sync_copy(dinv, dinvb)
    pltpu.sync_copy(pstack.at[0], p1b)
    pltpu.sync_copy(pstack.at[1], p2b)

    @pl.loop(0, WCH)
    def _(j):
        for g in range(8):
            sl = pl.ds(g * 16, 16)
            s16 = srcb[j, sl]
            d16 = dstb[j, sl]
            w16 = ewb[j, sl]
            ewb[j, sl] = (w16 * plsc.load_gather(dinvb, [s16])
                          * plsc.load_gather(dinvb, [d16]))
            p1o[j, sl] = plsc.load_gather(p1b, [s16])
            p2o[j, sl] = plsc.load_gather(p2b, [s16])

    pltpu.sync_copy(ewb, normd.at[w])
    pltpu.sync_copy(p1o, pidxd.at[w])
    pltpu.sync_copy(p2o, pidxd.at[32 + w])


_norm_call = pl.kernel(
    _norm_body,
    out_type=[
        jax.ShapeDtypeStruct((32, WCH, C), _f32),
        jax.ShapeDtypeStruct((64, WCH, C), _i32),
    ],
    mesh=_mesh,
    scratch_types=[
        pltpu.VMEM((WCH, C), _i32),
        pltpu.VMEM((WCH, C), _i32),
        pltpu.VMEM((WCH, C), _f32),
        pltpu.VMEM((WCH, C), _i32),
        pltpu.VMEM((WCH, C), _i32),
        pltpu.VMEM((N,), _f32),
        pltpu.VMEM((N,), _i32),
        pltpu.VMEM((N,), _i32),
    ],
    compiler_params=_sc_params,
)


def _conv_body(srcd, dstd, nrmd, pidxd, tab, zeros2,
               zA, zB,
               ra0, ra1,
               sw0, sw1, sw2, sw3, dw0, dw1, dw2, dw3,
               nw0, nw1, nw2, nw3,
               acc, gsem0, gsem1, ssem0, ssem1,
               wsem0, wsem1, wsem2, wsem3):
    c = lax.axis_index("c")
    s = lax.axis_index("s")
    off = c * N
    ra = (ra0, ra1)
    sw = (sw0, sw1, sw2, sw3)
    dw = (dw0, dw1, dw2, dw3)
    nw = (nw0, nw1, nw2, nw3)
    gsem = (gsem0, gsem1)
    ssem = (ssem0, ssem1)
    wsem = (wsem0, wsem1, wsem2, wsem3)

    def zero_acc():
        pltpu.sync_copy(zeros2.at[pl.ds(s * SB, SB)],
                        acc.at[pl.ds(s * SB, SB)])

        @pl.when(s == NT - 1)
        def _():
            pltpu.sync_copy(zeros2.at[pl.ds(SB * NT, SREM)],
                            acc.at[pl.ds(SB * NT, SREM)])

    def stripe_out(o):
        pltpu.sync_copy(acc.at[pl.ds(s * SB, SB)],
                        o.at[pl.ds(c * N + s * SB, SB)])

        @pl.when(s == NT - 1)
        def _():
            pltpu.sync_copy(acc.at[pl.ds(SB * NT, SREM)],
                            o.at[pl.ds(c * N + SB * NT, SREM)])

    def scale(jn, b, wi):
        @pl.loop(0, C)
        def _(r):
            rv = jnp.full((16,), r, _i32)
            nb = plsc.load_gather(nw[wi], [rv])
            for q in range(8):
                sl = pl.ds(q * 16, 16)
                ra[b][r, sl] = ra[b][r, sl] * nb

    def make_pass(permuted, out):
        idxsrc = pidxd if permuted else srcd
        slab = (c * NT + s) if permuted else s

        def issue_windows(jn, wi):
            pltpu.async_copy(idxsrc.at[slab, jn], sw[wi], wsem[wi])
            pltpu.async_copy(dstd.at[s, jn], dw[wi], wsem[wi])
            pltpu.async_copy(nrmd.at[s, jn], nw[wi], wsem[wi])

        def wait_windows(wi):
            pltpu.make_async_copy(idxsrc.at[slab, 0], sw[wi], wsem[wi]).wait()
            pltpu.make_async_copy(dstd.at[s, 0], dw[wi], wsem[wi]).wait()
            pltpu.make_async_copy(nrmd.at[s, 0], nw[wi], wsem[wi]).wait()

        def add_off(wi):
            for g in range(8):
                sl = pl.ds(g * 16, 16)
                sw[wi][sl] = sw[wi][sl] + off

        def issue_gather(b, wi):
            pltpu.async_copy(tab.at[sw[wi]], ra[b], gsem[b])

        def wait_gather(b, wi):
            pltpu.make_async_copy(tab.at[sw[wi]], ra[b], gsem[b]).wait()

        def issue_scatter(b, wi):
            pltpu.async_copy(ra[b], acc.at[dw[wi]], ssem[b], add=True)

        def wait_scatter(b):
            pltpu.make_async_copy(ra[b], acc.at[dw[0]], ssem[b]).wait()

        issue_windows(0, 0)
        issue_windows(1, 1)
        wait_windows(0)
        add_off(0)
        issue_gather(0, 0)

        @pl.loop(0, NCHUNK, step=4)
        def _(j0):
            for u in range(4):
                j = j0 + u
                b = u % 2
                b1 = (u + 1) % 2
                w4 = u
                w41 = (u + 1) % 4
                w42 = (u + 2) % 4
                wait_gather(b, w4)

                @pl.when(j + 1 < NCHUNK)
                def _():
                    @pl.when(j > 0)
                    def _():
                        wait_scatter(b1)

                    wait_windows(w41)
                    add_off(w41)
                    issue_gather(b1, w41)

                @pl.when(j + 2 < NCHUNK)
                def _():
                    issue_windows(j + 2, w42)

                scale(j, b, w4)
                issue_scatter(b, w4)

        wait_scatter(1)
        plsc.subcore_barrier()
        stripe_out(out)

    zero_acc()
    plsc.subcore_barrier()
    make_pass(False, zA)
    zero_acc()
    plsc.subcore_barrier()
    make_pass(True, zB)


_conv_call = pl.kernel(
    _conv_body,
    out_type=[jax.ShapeDtypeStruct((2 * N, D), _f32) for _ in range(2)],
    mesh=_mesh,
    scratch_types=(
        [pltpu.VMEM((C, D), _f32) for _ in range(2)]
        + [pltpu.VMEM((C,), _i32) for _ in range(8)]
        + [pltpu.VMEM((C,), _f32) for _ in range(4)]
        + [pltpu.VMEM_SHARED((N, D), _f32)]
        + [pltpu.SemaphoreType.DMA for _ in range(8)]
    ),
    compiler_params=_sc_params,
)


_FB = 2000
_FSTEPS = N // _FB


def _fin_body(zA_ref, zB_ref, wp_ref, bp_ref,
              o1, o2, og1, og2, o1n, o2n, s1, s2):
    i = pl.program_id(0)

    blkA = zA_ref[...]
    blkB = zB_ref[...]
    r1 = jnp.maximum(blkA[0], 0.0)
    r2 = jnp.maximum(blkA[1], 0.0)
    o1[...] = r1
    o2[...] = r2
    o1n[...] = jnp.maximum(blkB[0], 0.0)
    o2n[...] = jnp.maximum(blkB[1], 0.0)

    prev1 = jnp.where(i == 0, jnp.zeros_like(s1[...]), s1[...])
    prev2 = jnp.where(i == 0, jnp.zeros_like(s2[...]), s2[...])
    s1[...] = prev1 + jnp.sum(r1, axis=0, keepdims=True)
    s2[...] = prev2 + jnp.sum(r2, axis=0, keepdims=True)

    @pl.when(i == _FSTEPS - 1)
    def _():
        wp_t = wp_ref[...].T
        og1[...] = jax.nn.sigmoid(s1[...] / N) @ wp_t + bp_ref[...]
        og2[...] = jax.nn.sigmoid(s2[...] / N) @ wp_t + bp_ref[...]


def _fin_call(zA, zB, wp, bp2):
    zspec = pl.BlockSpec((2, _FB, D), lambda i: (0, i, 0))
    ospec = pl.BlockSpec((_FB, D), lambda i: (i, 0))
    gspec = pl.BlockSpec((1, D), lambda i: (0, 0))
    return pl.pallas_call(
        _fin_body,
        grid=(_FSTEPS,),
        in_specs=[zspec, zspec,
                  pl.BlockSpec((D, D), lambda i: (0, 0)),
                  pl.BlockSpec((1, D), lambda i: (0, 0))],
        out_specs=[ospec, ospec, gspec, gspec, ospec, ospec],
        out_shape=[
            jax.ShapeDtypeStruct((N, D), _f32),
            jax.ShapeDtypeStruct((N, D), _f32),
            jax.ShapeDtypeStruct((1, D), _f32),
            jax.ShapeDtypeStruct((1, D), _f32),
            jax.ShapeDtypeStruct((N, D), _f32),
            jax.ShapeDtypeStruct((N, D), _f32),
        ],
        scratch_shapes=[pltpu.VMEM((1, D), _f32), pltpu.VMEM((1, D), _f32)],
    )(zA, zB, wp, bp2)


def kernel(x, edge_index, edge_weight, W1, W2, Wp, bp, perm1, perm2):
    src = edge_index[0].astype(_i32)
    dst = edge_index[1].astype(_i32)
    ew = edge_weight.astype(_f32)
    pad_idx = (jnp.arange(EPAD - E, dtype=_i32) * 37) % N
    src_p = jnp.concatenate([src, pad_idx])
    dst_p = jnp.concatenate([dst, pad_idx])
    ew_p = jnp.concatenate([ew, jnp.zeros((EPAD - E,), _f32)])
    srcd32 = src_p.reshape(32, WCH, C)
    dstd32 = dst_p.reshape(32, WCH, C)
    ewd32 = ew_p.reshape(32, WCH, C)

    degp = _deg_call(dstd32, ewd32, jnp.zeros((N,), _f32))
    tab = _mm_call(x, jnp.stack([W1, W2]))
    dinv = _dinv_call(degp.reshape(2, N)).reshape(N)

    pstack = jnp.stack([perm1.astype(_i32), perm2.astype(_i32)])
    normd, pidxd = _norm_call(srcd32, dstd32, ewd32, dinv, pstack)

    zA, zB = _conv_call(
        src_p.reshape(NT, NCHUNK, C), dst_p.reshape(NT, NCHUNK, C),
        normd.reshape(NT, NCHUNK, C), pidxd.reshape(2 * NT, NCHUNK, C),
        tab, jnp.zeros((N, D), _f32))

    z1, z2, g1, g2, z1n, z2n = _fin_call(
        zA.reshape(2, N, D), zB.reshape(2, N, D), Wp, bp.reshape(1, D))
    return z1, z2, g1, g2, z1n, z2n
